# Initial kernel scaffold; baseline (speedup 1.0000x reference)
import jax
import jax.numpy as jnp
from jax.experimental import pallas as pl


def kernel(action_idx, is_ground, physics_params, action_emb, W1, b1, W2, b2, W3, b3, gravity):
    raise NotImplementedError("write your pallas kernel here")


# speedup vs baseline: 2.2798x; 2.2798x over previous
import functools

import jax
import jax.numpy as jnp
from jax import lax
from jax.experimental import pallas as pl
from jax.experimental.pallas import tpu as pltpu
from jax.experimental.pallas import tpu_sc as plsc

BATCH = 16384
FEAT = 16


def _sc_gather(table, idx):
    info = plsc.get_sparse_core_info()
    nw = info.num_cores * info.num_subcores
    b_per_w = BATCH // nw
    mesh = plsc.VectorSubcoreMesh(core_axis_name="c", subcore_axis_name="s")

    @functools.partial(
        pl.kernel,
        mesh=mesh,
        compiler_params=pltpu.CompilerParams(use_tc_tiling_on_sc=False),
        out_type=jax.ShapeDtypeStruct((BATCH, FEAT), jnp.float32),
        scratch_types=[
            pltpu.VMEM((b_per_w,), jnp.int32),
            pltpu.VMEM((b_per_w, FEAT), jnp.float32),
            pltpu.SemaphoreType.DMA,
        ],
    )
    def gather_k(table_hbm, idx_hbm, out_hbm, idx_v, rows_v, sem):
        wid = lax.axis_index("s") * info.num_cores + lax.axis_index("c")
        base = wid * b_per_w
        pltpu.sync_copy(idx_hbm.at[pl.ds(base, b_per_w)], idx_v)
        pltpu.async_copy(table_hbm.at[idx_v], rows_v, sem).wait()
        pltpu.sync_copy(rows_v, out_hbm.at[pl.ds(base, b_per_w)])

    return gather_k(table, idx)


def _tc_mlp(xg, ig, w1p, wig, b1, w2t, b2, w3t, b3):
    blk = 2048
    grid = BATCH // blk

    def body(x_ref, ig_ref, w1_ref, wig_ref, b1_ref, w2_ref, b2_ref, w3_ref,
             b3_ref, o_ref):
        x = x_ref[:]
        g = ig_ref[:]
        h = jnp.dot(x, w1_ref[:], preferred_element_type=jnp.float32)
        h = jnp.maximum(h + g * wig_ref[:] + b1_ref[:], 0.0)
        h = jnp.dot(h, w2_ref[:], preferred_element_type=jnp.float32)
        h = jnp.maximum(h + b2_ref[:], 0.0)
        r = jnp.dot(h, w3_ref[:], preferred_element_type=jnp.float32)
        o_ref[:] = x[:, 0:2] + r + b3_ref[:]

    full = lambda shape: pl.BlockSpec(shape, lambda i: (0, 0))
    return pl.pallas_call(
        body,
        grid=(grid,),
        in_specs=[
            pl.BlockSpec((blk, FEAT), lambda i: (i, 0)),
            pl.BlockSpec((blk, 1), lambda i: (i, 0)),
            full((FEAT, 32)),
            full((1, 32)),
            full((1, 32)),
            full((32, 16)),
            full((1, 16)),
            full((16, 2)),
            full((1, 2)),
        ],
        out_specs=pl.BlockSpec((blk, 2), lambda i: (i, 0)),
        out_shape=jax.ShapeDtypeStruct((BATCH, 2), jnp.float32),
    )(xg, ig, w1p, wig, b1, w2t, b2, w3t, b3)


def kernel(action_idx, is_ground, physics_params, action_emb, W1, b1, W2, b2,
           W3, b3, gravity):
    idx = action_idx.astype(jnp.int32)
    n = physics_params.shape[0]
    table = jnp.concatenate(
        [physics_params[:, :2], action_emb,
         jnp.zeros((n, FEAT - 10), jnp.float32)], axis=1)
    w1p = jnp.zeros((FEAT, 32), jnp.float32).at[2:10].set(W1[:, :8].T)
    wig = W1[:, 8].reshape(1, 32)
    xg = _sc_gather(table, idx)
    out = _tc_mlp(xg, is_ground.reshape(BATCH, 1), w1p, wig,
                  b1.reshape(1, 32), W2.T, b2.reshape(1, 16), W3.T,
                  b3.reshape(1, 2))
    return (out, gravity)
